# baseline (speedup 1.0000x reference)
import functools
import jax
import jax.numpy as jnp
from jax import lax
from jax.experimental import pallas as pl
from jax.experimental.pallas import tpu as pltpu
from jax.experimental.pallas import tpu_sc as plsc

U0 = 10.0
R = 0.2

_N = 65536
_M = 64
_NC = 2
_NS = 16
_NW = _NC * _NS
_PER_W = _N // _NW
_NV = _PER_W // 16


def _nsqrt(m):
    m = jnp.maximum(m, jnp.float32(1e-30))
    i = lax.bitcast_convert_type(m, jnp.int32)
    i = jnp.int32(0x5F3759DF) - (i >> 1)
    y = lax.bitcast_convert_type(i, jnp.float32)
    for _ in range(3):
        y = y * (jnp.float32(1.5) - jnp.float32(0.5) * m * y * y)
    return m * y


def kernel(state, B0, B1):
    bsplat = jnp.repeat(
        jnp.stack([B0[:, 0], B0[:, 1], B1[:, 0], B1[:, 1]], axis=0),
        16, axis=1).reshape(4 * _M * 16)
    xs = state[:, 0]
    ys = state[:, 1]

    mesh = plsc.VectorSubcoreMesh(core_axis_name="c", subcore_axis_name="s")

    @functools.partial(
        pl.kernel,
        mesh=mesh,
        out_type=[jax.ShapeDtypeStruct((_N,), jnp.float32)] * 2,
        scratch_types=[
            pltpu.VMEM((_PER_W,), jnp.float32),
            pltpu.VMEM((_PER_W,), jnp.float32),
            pltpu.VMEM((4 * _M * 16,), jnp.float32),
            pltpu.VMEM((_PER_W,), jnp.float32),
            pltpu.VMEM((_PER_W,), jnp.float32),
        ],
    )
    def k(xs_hbm, ys_hbm, bs_hbm, o0_hbm, o1_hbm, x_v, y_v, bs_v, o0_v, o1_v):
        wid = lax.axis_index("s") * _NC + lax.axis_index("c")
        base = wid * _PER_W
        pltpu.sync_copy(xs_hbm.at[pl.ds(base, _PER_W)], x_v)
        pltpu.sync_copy(ys_hbm.at[pl.ds(base, _PER_W)], y_v)
        pltpu.sync_copy(bs_hbm, bs_v)

        def body(i, carry):
            sl = pl.ds(i * 16, 16)
            xv = x_v[sl]
            yv = y_v[sl]

            def min_d2(xrow, yrow):
                m = None
                for j in range(_M):
                    dx = xv - bs_v[pl.ds(xrow * (_M * 16) + 16 * j, 16)]
                    dy = yv - bs_v[pl.ds(yrow * (_M * 16) + 16 * j, 16)]
                    d2 = dx * dx + dy * dy
                    m = d2 if m is None else jnp.minimum(m, d2)
                return m

            o0_v[sl] = U0 * jnp.exp(-_nsqrt(min_d2(0, 1)) / R)
            o1_v[sl] = U0 * jnp.exp(-_nsqrt(min_d2(2, 3)) / R)
            return carry

        lax.fori_loop(0, _NV, body, jnp.int32(0))
        pltpu.sync_copy(o0_v, o0_hbm.at[pl.ds(base, _PER_W)])
        pltpu.sync_copy(o1_v, o1_hbm.at[pl.ds(base, _PER_W)])

    o0, o1 = k(xs, ys, bsplat)
    return jnp.stack([o0, o1], axis=1)


# speedup vs baseline: 1.1645x; 1.1645x over previous
import jax
import jax.numpy as jnp
from jax import lax
from jax.experimental import pallas as pl

U0 = 10.0
R = 0.2

_N = 65536
_M = 64
_BN = 4096


def _ped_kernel(w_ref, p_ref, o0_ref, o1_ref):
    p = p_ref[...]
    lane = lax.broadcasted_iota(jnp.int32, p.shape, 1)
    q = jnp.where(lane == 2, jnp.float32(1.0), p)
    mt = lax.dot_general(w_ref[...], q, (((1,), (1,)), ((), ())),
                         preferred_element_type=jnp.float32)
    lin0 = jnp.min(mt[0:_M], axis=0, keepdims=True)
    lin1 = jnp.min(mt[_M:2 * _M], axis=0, keepdims=True)
    xr = mt[128:129]
    yr = mt[129:130]
    r2 = xr * xr + yr * yr
    d0 = jnp.maximum(lin0 + r2, 0.0)
    d1 = jnp.maximum(lin1 + r2, 0.0)
    o0_ref[...] = (U0 * jnp.exp(-jnp.sqrt(d0) / R)).reshape(_BN // 128, 128)
    o1_ref[...] = (U0 * jnp.exp(-jnp.sqrt(d1) / R)).reshape(_BN // 128, 128)


def kernel(state, B0, B1):
    bx = jnp.concatenate([B0[:, 0], B1[:, 0]])
    by = jnp.concatenate([B0[:, 1], B1[:, 1]])
    cj = bx * bx + by * by
    w = jnp.zeros((136, 4), jnp.float32)
    w = w.at[:128, 0].set(-2.0 * bx)
    w = w.at[:128, 1].set(-2.0 * by)
    w = w.at[:128, 2].set(cj)
    w = w.at[128, 0].set(1.0)
    w = w.at[129, 1].set(1.0)

    rows = _BN // 128
    o0, o1 = pl.pallas_call(
        _ped_kernel,
        grid=(_N // _BN,),
        in_specs=[pl.BlockSpec((136, 4), lambda i: (0, 0)),
                  pl.BlockSpec((_BN, 4), lambda i: (i, 0))],
        out_specs=[pl.BlockSpec((rows, 128), lambda i: (i, 0))] * 2,
        out_shape=[jax.ShapeDtypeStruct((512, 128), jnp.float32)] * 2,
    )(w, state)
    return jnp.stack([o0.reshape(-1), o1.reshape(-1)], axis=1)


# speedup vs baseline: 1.3080x; 1.1233x over previous
import jax
import jax.numpy as jnp
from jax.experimental import pallas as pl
from jax.experimental.pallas import tpu as pltpu

U0 = 10.0
R = 0.2

_N = 65536
_M = 64
_BN = 4096
_RW = _BN // 128


def _ped_kernel(b0_ref, b1_ref, xy_ref, out_ref):
    xy = xy_ref[...]
    x = xy[0:1].reshape(_RW, 128)
    y = xy[1:2].reshape(_RW, 128)

    def min_d2(b_ref):
        m = None
        for j in range(_M):
            dx = x - b_ref[j, 0]
            dy = y - b_ref[j, 1]
            d2 = dx * dx + dy * dy
            m = d2 if m is None else jnp.minimum(m, d2)
        return m

    o0 = U0 * jnp.exp(-jnp.sqrt(min_d2(b0_ref)) / R)
    o1 = U0 * jnp.exp(-jnp.sqrt(min_d2(b1_ref)) / R)
    o = jnp.concatenate([o0.reshape(1, _BN), o1.reshape(1, _BN)], axis=0)
    out_ref[...] = o.T


def kernel(state, B0, B1):
    xy = state[:, 0:2].T
    smem = pl.BlockSpec(memory_space=pltpu.SMEM)
    return pl.pallas_call(
        _ped_kernel,
        grid=(_N // _BN,),
        in_specs=[smem, smem,
                  pl.BlockSpec((2, _BN), lambda i: (0, i))],
        out_specs=pl.BlockSpec((_BN, 2), lambda i: (i, 0)),
        out_shape=jax.ShapeDtypeStruct((_N, 2), jnp.float32),
    )(B0, B1, xy)
